# baseline (speedup 1.0000x reference)
import jax
import jax.numpy as jnp
from jax import lax
from jax.experimental import pallas as pl
from jax.experimental.pallas import tpu as pltpu
from jax.experimental.pallas import tpu_sc as plsc

WORDS_SIZE = 1000000
SENS_LEN = 200
EMBEDS_DIM = 64
BATCH = 4096

NUM_WORKERS = 32
SENS_PER_WORKER = BATCH // NUM_WORKERS
CHUNK_SENS = 2
CHUNK_ROWS = CHUNK_SENS * SENS_LEN
CHUNKS_PER_WORKER = SENS_PER_WORKER // CHUNK_SENS
VREGS_PER_ROW = EMBEDS_DIM // 16


def _positional_encoding():
    pos = jnp.arange(SENS_LEN, dtype=jnp.float32)[:, None]
    i = jnp.arange(EMBEDS_DIM, dtype=jnp.float32)[None, :]
    pe_val = pos / jnp.power(10000.0, i / EMBEDS_DIM)
    return jnp.where((jnp.arange(EMBEDS_DIM)[None, :] % 2) == 0,
                     jnp.sin(pe_val), jnp.cos(pe_val))


def _sc_body(table, x, pe, out, pe_v, idx0, idx1, rows0, rows1,
             sem0, sem1, sem2, sem3):
    wid = lax.axis_index("s") * 2 + lax.axis_index("c")
    chunk_base = wid * CHUNKS_PER_WORKER

    pltpu.sync_copy(pe, pe_v)

    idx = [idx0, idx1]
    rows = [rows0, rows1]
    sem = [(sem0, sem1), (sem2, sem3)]

    def load_and_fire(c, b):
        pltpu.sync_copy(x.at[pl.ds(c * CHUNK_ROWS, CHUNK_ROWS)], idx[b])

        def row_gather(t, carry):
            vec = idx[b][pl.ds(t * 16, 16)]
            for j in range(16):
                v = vec[j]
                pltpu.async_copy(table.at[pl.ds(v, 1)],
                                 rows[b].at[pl.ds(t * 16 + j, 1)],
                                 sem[b][j % 2])
            return carry

        lax.fori_loop(0, CHUNK_ROWS // 16, row_gather, 0)

    def process(c, b):
        for q in range(2):
            pltpu.make_async_copy(table.at[pl.ds(0, CHUNK_ROWS // 2)],
                                  rows[b].at[pl.ds(0, CHUNK_ROWS // 2)],
                                  sem[b][q]).wait()

        def sbody(s, carry):
            for cc in range(CHUNK_SENS):
                r = cc * SENS_LEN + s
                for k in range(VREGS_PER_ROW):
                    sl = pl.ds(k * 16, 16)
                    rows[b][r, sl] = rows[b][r, sl] + pe_v[s, sl]
            return carry

        lax.fori_loop(0, SENS_LEN, sbody, 0)
        for cc in range(CHUNK_SENS):
            pltpu.sync_copy(rows[b].at[pl.ds(cc * SENS_LEN, SENS_LEN)],
                            out.at[c * CHUNK_SENS + cc])

    load_and_fire(chunk_base, 0)

    def outer(i, carry):
        c0 = chunk_base + 2 * i
        load_and_fire(c0 + 1, 1)
        process(c0, 0)
        load_and_fire(c0 + 2, 0)
        process(c0 + 1, 1)
        return carry

    lax.fori_loop(0, CHUNKS_PER_WORKER // 2 - 1, outer, 0)

    last = chunk_base + CHUNKS_PER_WORKER - 1
    load_and_fire(last, 1)
    process(last - 1, 0)
    process(last, 1)


@jax.jit
def kernel(x, table):
    pe = _positional_encoding()
    xi = x.astype(jnp.int32).reshape(-1)

    mesh = plsc.VectorSubcoreMesh(core_axis_name="c", subcore_axis_name="s")
    out = pl.kernel(
        _sc_body,
        out_type=jax.ShapeDtypeStruct((BATCH, SENS_LEN, EMBEDS_DIM),
                                      jnp.float32),
        mesh=mesh,
        scratch_types=[
            pltpu.VMEM((SENS_LEN, EMBEDS_DIM), jnp.float32),
            pltpu.VMEM((CHUNK_ROWS,), jnp.int32),
            pltpu.VMEM((CHUNK_ROWS,), jnp.int32),
            pltpu.VMEM((CHUNK_ROWS, EMBEDS_DIM), jnp.float32),
            pltpu.VMEM((CHUNK_ROWS, EMBEDS_DIM), jnp.float32),
            pltpu.SemaphoreType.DMA,
            pltpu.SemaphoreType.DMA,
            pltpu.SemaphoreType.DMA,
            pltpu.SemaphoreType.DMA,
        ],
    )(table, xi, pe)
    return out
